# baseline (speedup 1.0000x reference)
import functools

import jax
import jax.numpy as jnp
from jax import lax
from jax.experimental import pallas as pl
from jax.experimental.pallas import tpu as pltpu
from jax.experimental.pallas import tpu_sc as plsc

INPUT_DIM = 512
NUM_NEURONS = 512
BATCH = 1024
TIMESTEPS = 100

_NC = 2
_NS = 16
_NW = _NC * _NS
_ROWS = BATCH // _NW
_NSL = INPUT_DIM // 16


def _body(x_hbm, out_hbm, x_v, buf, strow):
    wid = lax.axis_index("s") * _NC + lax.axis_index("c")
    base = wid * _ROWS
    pltpu.sync_copy(x_hbm.at[pl.ds(base, _ROWS)], x_v)

    zero_f = jnp.zeros((16,), jnp.float32)
    one_f = jnp.ones((16,), jnp.float32)
    zero_i = jnp.zeros((16,), jnp.int32)
    lane = lax.iota(jnp.int32, 16)

    def _zb(i, _):
        buf[i // _NSL, pl.ds((i % _NSL) * 16, 16)] = zero_f
        return 0

    lax.fori_loop(0, TIMESTEPS * _NSL, _zb, 0)

    def _zs(j, _):
        strow[pl.ds(j * 16, 16)] = zero_i
        return 0

    lax.fori_loop(0, _NSL, _zs, 0)

    def _row(r, _):
        def _slice(j, _):
            col = lane + j * 16
            old = strow[pl.ds(j * 16, 16)]
            plsc.store_scatter(buf, [old, col], zero_f)
            xs = x_v[r, pl.ds(j * 16, 16)]
            s = 1.0 / (1.0 + jnp.exp(-xs))
            st = (s * jnp.float32(TIMESTEPS - 1)).astype(jnp.int32)
            plsc.store_scatter(buf, [st, col], one_f)
            strow[pl.ds(j * 16, 16)] = st
            return 0

        lax.fori_loop(0, _NSL, _slice, 0)
        pltpu.sync_copy(buf, out_hbm.at[:, base + r, :])
        return 0

    lax.fori_loop(0, _ROWS, _row, 0)


def kernel(continuous_input, timesteps):
    del timesteps
    mesh = plsc.VectorSubcoreMesh(core_axis_name="c", subcore_axis_name="s")
    run = pl.kernel(
        _body,
        out_type=jax.ShapeDtypeStruct((TIMESTEPS, BATCH, NUM_NEURONS), jnp.float32),
        mesh=mesh,
        scratch_types=[
            pltpu.VMEM((_ROWS, INPUT_DIM), jnp.float32),
            pltpu.VMEM((TIMESTEPS, NUM_NEURONS), jnp.float32),
            pltpu.VMEM((INPUT_DIM,), jnp.int32),
        ],
        compiler_params=pltpu.CompilerParams(
            use_tc_tiling_on_sc=False, needs_layout_passes=False
        ),
    )
    return run(continuous_input)
